# baseline (speedup 1.0000x reference)
import jax
import jax.numpy as jnp
from jax.experimental import pallas as pl
from jax.experimental.pallas import tpu as pltpu

BM = 400
H = BM // 2


def _mm_kernel(a0_ref, a1_ref, b_ref, o_ref):
    b = b_ref[...]
    o_ref[:H, :] = jnp.dot(
        a0_ref[...], b,
        preferred_element_type=jnp.float32,
        precision=jax.lax.Precision.DEFAULT,
    )
    o_ref[H:, :] = jnp.dot(
        a1_ref[...], b,
        preferred_element_type=jnp.float32,
        precision=jax.lax.Precision.DEFAULT,
    )


def kernel(adj, embeds):
    m, kdim = adj.shape
    _, n = embeds.shape
    return pl.pallas_call(
        _mm_kernel,
        grid=(m // BM,),
        in_specs=[
            pl.BlockSpec((H, kdim), lambda i: (2 * i, 0)),
            pl.BlockSpec((H, kdim), lambda i: (2 * i + 1, 0)),
            pl.BlockSpec((kdim, n), lambda i: (0, 0)),
        ],
        out_specs=pl.BlockSpec((BM, n), lambda i: (i, 0)),
        out_shape=jax.ShapeDtypeStruct((m, n), jnp.float32),
        compiler_params=pltpu.CompilerParams(
            dimension_semantics=("parallel",),
            vmem_limit_bytes=64 * 1024 * 1024,
        ),
    )(adj, adj, embeds)


# speedup vs baseline: 1.0144x; 1.0144x over previous
import jax
import jax.numpy as jnp
from jax.experimental import pallas as pl
from jax.experimental.pallas import tpu as pltpu

BM = 400
HN = 256


def _mm_kernel(a_ref, b0_ref, b1_ref, o_ref):
    a = a_ref[...]
    o_ref[:, :HN] = jnp.dot(
        a, b0_ref[...],
        preferred_element_type=jnp.float32,
        precision=jax.lax.Precision.DEFAULT,
    )
    o_ref[:, HN:] = jnp.dot(
        a, b1_ref[...],
        preferred_element_type=jnp.float32,
        precision=jax.lax.Precision.DEFAULT,
    )


def kernel(adj, embeds):
    m, kdim = adj.shape
    _, n = embeds.shape
    return pl.pallas_call(
        _mm_kernel,
        grid=(m // BM,),
        in_specs=[
            pl.BlockSpec((BM, kdim), lambda i: (i, 0)),
            pl.BlockSpec((kdim, HN), lambda i: (0, 0)),
            pl.BlockSpec((kdim, HN), lambda i: (0, 1)),
        ],
        out_specs=pl.BlockSpec((BM, n), lambda i: (i, 0)),
        out_shape=jax.ShapeDtypeStruct((m, n), jnp.float32),
        compiler_params=pltpu.CompilerParams(
            dimension_semantics=("parallel",),
            vmem_limit_bytes=64 * 1024 * 1024,
        ),
    )(adj, embeds, embeds)


# speedup vs baseline: 1.0151x; 1.0007x over previous
import jax
import jax.numpy as jnp
from jax.experimental import pallas as pl
from jax.experimental.pallas import tpu as pltpu

BM = 400


def _mm_kernel(a_ref, b_ref, o_ref):
    o_ref[...] = jnp.dot(
        a_ref[...], b_ref[...],
        preferred_element_type=jnp.float32,
        precision=jax.lax.Precision.DEFAULT,
    )


def kernel(adj, embeds):
    m, kdim = adj.shape
    _, n = embeds.shape
    return pl.pallas_call(
        _mm_kernel,
        grid=(m // BM,),
        in_specs=[
            pl.BlockSpec((BM, kdim), lambda i: (i, 0)),
            pl.BlockSpec((kdim, n), lambda i: (0, 0)),
        ],
        out_specs=pl.BlockSpec((BM, n), lambda i: (i, 0)),
        out_shape=jax.ShapeDtypeStruct((m, n), jnp.float32),
        compiler_params=pltpu.CompilerParams(
            dimension_semantics=("parallel",),
        ),
    )(adj, embeds)
